# baseline (speedup 1.0000x reference)
import functools

import jax
import jax.numpy as jnp
from jax import lax
from jax.experimental import pallas as pl
from jax.experimental.pallas import tpu as pltpu
from jax.experimental.pallas import tpu_sc as plsc

VOCAB = 100000
D = 128
HOPS = 3
B, M, L, S = 128, 1024, 4, 200

NW = 32
CHUNK_O = 32
CHUNK_I = CHUNK_O * L
ROWS_PER_W = (B * M) // NW
N_CHUNKS = ROWS_PER_W // CHUNK_O


NT = HOPS + 1


def _sc_embed(mem_flat, c_tables):
    mesh = plsc.VectorSubcoreMesh(core_axis_name="c", subcore_axis_name="s")

    @functools.partial(
        pl.kernel,
        out_type=jax.ShapeDtypeStruct((B, NT, M, D), jnp.float32),
        mesh=mesh,
        scratch_types=[
            pltpu.VMEM((2, CHUNK_I), jnp.int32),
            pltpu.VMEM((2, CHUNK_I, D), jnp.float32),
            pltpu.VMEM((2, NT, CHUNK_O, D), jnp.float32),
            pltpu.SemaphoreType.DMA,
            pltpu.SemaphoreType.DMA,
            pltpu.SemaphoreType.DMA,
            pltpu.SemaphoreType.DMA,
            pltpu.SemaphoreType.DMA,
            pltpu.SemaphoreType.DMA,
        ],
    )
    def k(mem_hbm, c_hbm, out_hbm, idx_v, rows_v, out_v,
          isem0, isem1, gsem0, gsem1, osem0, osem1):
        wid = lax.axis_index("s") * 2 + lax.axis_index("c")
        row0 = wid * ROWS_PER_W
        isems = (isem0, isem1)
        gsems = (gsem0, gsem1)
        osems = (osem0, osem1)

        def idx_load(ci, buf, sem):
            return pltpu.async_copy(
                mem_hbm.at[pl.ds((row0 + ci * CHUNK_O) * L, CHUNK_I)],
                idx_v.at[buf], sem)

        def gather(cbuf, tk, gbuf):
            return pltpu.async_copy(c_hbm.at[tk].at[idx_v.at[cbuf]],
                                    rows_v.at[gbuf], gsems[gbuf])

        def wait_gather(gbuf):
            pltpu.make_async_copy(c_hbm.at[0].at[idx_v.at[0]],
                                  rows_v.at[gbuf], gsems[gbuf]).wait()

        idx_load(0, 0, isems[0]).wait()
        gather(0, 0, 0)

        def pair_body(cj, carry):
            for par in range(2):
                ci = 2 * cj + par
                out_row = row0 + ci * CHUNK_O
                b = out_row // M
                m = lax.rem(out_row, M)
                obuf = par
                cbuf = par

                @pl.when(ci + 1 < N_CHUNKS)
                def _(ci=ci, cbuf=cbuf):
                    idx_load(ci + 1, 1 - cbuf, isems[1 - cbuf])

                @pl.when(ci >= 2)
                def _(obuf=obuf):
                    for tk in range(NT):
                        pltpu.make_async_copy(
                            out_v.at[obuf, tk],
                            out_hbm.at[0, tk, pl.ds(0, CHUNK_O)],
                            osems[obuf]).wait()

                for tk in range(NT):
                    gbuf = tk % 2
                    if tk < NT - 1:
                        gather(cbuf, tk + 1, 1 - gbuf)
                    else:
                        @pl.when(ci + 1 < N_CHUNKS)
                        def _(cbuf=cbuf, gbuf=gbuf):
                            pltpu.make_async_copy(
                                mem_hbm.at[pl.ds(0, CHUNK_I)],
                                idx_v.at[1 - cbuf], isems[1 - cbuf]).wait()
                            gather(1 - cbuf, 0, 1 - gbuf)
                    wait_gather(gbuf)

                    def sumrow(i, acc, gbuf=gbuf, tk=tk, obuf=obuf):
                        for g in range(D // 16):
                            sl = pl.ds(g * 16, 16)
                            out_v[obuf, tk, i, sl] = (
                                rows_v[gbuf, L * i, sl] + rows_v[gbuf, L * i + 1, sl]
                                + rows_v[gbuf, L * i + 2, sl] + rows_v[gbuf, L * i + 3, sl])
                        return acc
                    lax.fori_loop(0, CHUNK_O, sumrow, 0, unroll=2)

                    pltpu.async_copy(out_v.at[obuf, tk],
                                     out_hbm.at[b, tk, pl.ds(m, CHUNK_O)],
                                     osems[obuf])
            return carry

        lax.fori_loop(0, N_CHUNKS // 2, pair_body, 0)

        for obuf in range(2):
            for tk in range(NT):
                pltpu.make_async_copy(
                    out_v.at[obuf, tk],
                    out_hbm.at[0, tk, pl.ds(0, CHUNK_O)],
                    osems[obuf]).wait()

    return k(mem_flat, c_tables)


def _tc_body(kb_ref, dl_ref, e_ref, dh_ref, q_ref, p2_ref, l2_ref):
    bi = pl.program_id(0)
    kbv = kb_ref[bi]
    dlv = dl_ref[bi]
    rows = lax.broadcasted_iota(jnp.int32, (M, S), 0)
    cols = lax.broadcasted_iota(jnp.int32, (M, S), 1)
    hi = lax.Precision.HIGHEST
    sel = jnp.where((rows == kbv + cols) & (cols < dlv), 1.0, 0.0).astype(jnp.float32)
    h_full = jnp.dot(sel, dh_ref[0], precision=hi,
                     preferred_element_type=jnp.float32)
    es = [e_ref[0, tk] + h_full for tk in range(HOPS + 1)]

    q = q_ref[0]
    dn = (((1,), (1,)), ((), ()))
    u = q
    logits = None
    for tk in range(HOPS):
        logits = lax.dot_general(u, es[tk], dn, precision=hi,
                                 preferred_element_type=jnp.float32)
        p = jax.nn.softmax(logits, axis=-1)
        u = u + jnp.dot(p, es[tk + 1], precision=hi, preferred_element_type=jnp.float32)
    gp = jax.nn.sigmoid(logits)

    u2 = q
    p2 = None
    l2 = None
    for tk in range(HOPS):
        l2 = lax.dot_general(u2, es[tk], dn, precision=hi,
                             preferred_element_type=jnp.float32) * gp
        p2 = jax.nn.softmax(l2, axis=-1)
        u2 = u2 + jnp.dot(p2 * gp, es[tk + 1], precision=hi,
                          preferred_element_type=jnp.float32)
    p2_ref[0] = p2
    l2_ref[0] = l2


def _tc_hops(kb, dl, e, dh, q, interpret=False):
    return pl.pallas_call(
        _tc_body,
        grid=(B,),
        in_specs=[
            pl.BlockSpec(memory_space=pltpu.SMEM),
            pl.BlockSpec(memory_space=pltpu.SMEM),
            pl.BlockSpec((1, NT, M, D), lambda b: (b, 0, 0, 0)),
            pl.BlockSpec((1, S, D), lambda b: (b, 0, 0)),
            pl.BlockSpec((1, 1, D), lambda b: (b, 0, 0)),
        ],
        out_specs=[pl.BlockSpec((1, 1, M), lambda b: (b, 0, 0)),
                   pl.BlockSpec((1, 1, M), lambda b: (b, 0, 0))],
        out_shape=[jax.ShapeDtypeStruct((B, 1, M), jnp.float32),
                   jax.ShapeDtypeStruct((B, 1, M), jnp.float32)],
        interpret=interpret,
    )(kb, dl, e, dh, q.reshape(B, 1, D))


def kernel(memory, kb_len, dialog_len, dialog_hidden, query, C):
    mem_flat = memory.reshape(-1).astype(jnp.int32)
    e = _sc_embed(mem_flat, C)
    probs2, logits2 = _tc_hops(kb_len.astype(jnp.int32), dialog_len.astype(jnp.int32),
                               e, dialog_hidden, query)
    return (probs2.reshape(B, M), logits2.reshape(B, M))


# speedup vs baseline: 1.3070x; 1.3070x over previous
import functools

import jax
import jax.numpy as jnp
from jax import lax
from jax.experimental import pallas as pl
from jax.experimental.pallas import tpu as pltpu
from jax.experimental.pallas import tpu_sc as plsc

VOCAB = 100000
D = 128
HOPS = 3
B, M, L, S = 128, 1024, 4, 200

NW = 32
CHUNK_O = 32
CHUNK_I = CHUNK_O * L
ROWS_PER_W = (B * M) // NW
N_CHUNKS = ROWS_PER_W // CHUNK_O


NT = HOPS + 1


def _sc_embed(mem_flat, c_tables):
    mesh = plsc.VectorSubcoreMesh(core_axis_name="c", subcore_axis_name="s")

    @functools.partial(
        pl.kernel,
        out_type=jax.ShapeDtypeStruct((B, NT, M, D), jnp.float32),
        mesh=mesh,
        scratch_types=[
            pltpu.VMEM((2, CHUNK_I), jnp.int32),
            pltpu.VMEM((2, CHUNK_I, D), jnp.float32),
            pltpu.VMEM((2, NT, CHUNK_O, D), jnp.float32),
            pltpu.SemaphoreType.DMA,
            pltpu.SemaphoreType.DMA,
            pltpu.SemaphoreType.DMA,
            pltpu.SemaphoreType.DMA,
            pltpu.SemaphoreType.DMA,
            pltpu.SemaphoreType.DMA,
        ],
    )
    def k(mem_hbm, c_hbm, out_hbm, idx_v, rows_v, out_v,
          isem0, isem1, gsem0, gsem1, osem0, osem1):
        wid = lax.axis_index("s") * 2 + lax.axis_index("c")
        row0 = wid * ROWS_PER_W
        isems = (isem0, isem1)
        gsems = (gsem0, gsem1)
        osems = (osem0, osem1)

        def idx_load(ci, buf, sem):
            return pltpu.async_copy(
                mem_hbm.at[pl.ds((row0 + ci * CHUNK_O) * L, CHUNK_I)],
                idx_v.at[buf], sem)

        def gather(cbuf, tk, gbuf):
            return pltpu.async_copy(c_hbm.at[tk].at[idx_v.at[cbuf]],
                                    rows_v.at[gbuf], gsems[gbuf])

        def wait_gather(gbuf):
            pltpu.make_async_copy(c_hbm.at[0].at[idx_v.at[0]],
                                  rows_v.at[gbuf], gsems[gbuf]).wait()

        idx_load(0, 0, isems[0]).wait()
        gather(0, 0, 0)

        def pair_body(cj, carry):
            for par in range(2):
                ci = 2 * cj + par
                out_row = row0 + ci * CHUNK_O
                b = out_row // M
                m = lax.rem(out_row, M)
                obuf = par
                cbuf = par

                @pl.when(ci + 1 < N_CHUNKS)
                def _(ci=ci, cbuf=cbuf):
                    idx_load(ci + 1, 1 - cbuf, isems[1 - cbuf])

                @pl.when(ci >= 2)
                def _(obuf=obuf):
                    for tk in range(NT):
                        pltpu.make_async_copy(
                            out_v.at[obuf, tk],
                            out_hbm.at[0, tk, pl.ds(0, CHUNK_O)],
                            osems[obuf]).wait()

                for tk in range(NT):
                    gbuf = tk % 2
                    if tk < NT - 1:
                        gather(cbuf, tk + 1, 1 - gbuf)
                    else:
                        @pl.when(ci + 1 < N_CHUNKS)
                        def _(cbuf=cbuf, gbuf=gbuf):
                            pltpu.make_async_copy(
                                mem_hbm.at[pl.ds(0, CHUNK_I)],
                                idx_v.at[1 - cbuf], isems[1 - cbuf]).wait()
                            gather(1 - cbuf, 0, 1 - gbuf)
                    wait_gather(gbuf)

                    def sumrow(i, acc, gbuf=gbuf, tk=tk, obuf=obuf):
                        for g in range(D // 16):
                            sl = pl.ds(g * 16, 16)
                            out_v[obuf, tk, i, sl] = (
                                rows_v[gbuf, L * i, sl] + rows_v[gbuf, L * i + 1, sl]
                                + rows_v[gbuf, L * i + 2, sl] + rows_v[gbuf, L * i + 3, sl])
                        return acc
                    lax.fori_loop(0, CHUNK_O, sumrow, 0, unroll=2)

                    pltpu.async_copy(out_v.at[obuf, tk],
                                     out_hbm.at[b, tk, pl.ds(m, CHUNK_O)],
                                     osems[obuf])
            return carry

        lax.fori_loop(0, N_CHUNKS // 2, pair_body, 0)

        for obuf in range(2):
            for tk in range(NT):
                pltpu.make_async_copy(
                    out_v.at[obuf, tk],
                    out_hbm.at[0, tk, pl.ds(0, CHUNK_O)],
                    osems[obuf]).wait()

    return k(mem_flat, c_tables)


def _tc_body(kb_ref, dl_ref, e_ref, dh_ref, q_ref, p2_ref, l2_ref):
    bi = pl.program_id(0)
    kbv = kb_ref[bi]
    dlv = dl_ref[bi]
    js = lax.broadcasted_iota(jnp.int32, (S, 1), 0)
    hid = dh_ref[0] * (js < dlv).astype(jnp.float32)
    for tk in range(NT):
        e_ref[0, tk, pl.ds(kbv, S)] = e_ref[0, tk, pl.ds(kbv, S)] + hid

    es_hi = []
    es_lo = []
    for tk in range(NT):
        ef = e_ref[0, tk]
        ehi = ef.astype(jnp.bfloat16)
        es_hi.append(ehi)
        es_lo.append((ef - ehi.astype(jnp.float32)).astype(jnp.bfloat16))

    def split(v):
        vhi = v.astype(jnp.bfloat16)
        return vhi, (v - vhi.astype(jnp.float32)).astype(jnp.bfloat16)

    dn = (((1,), (1,)), ((), ()))

    def dot_vE(v, tk):
        vhi, vlo = split(v)
        return (lax.dot_general(vhi, es_hi[tk], dn, preferred_element_type=jnp.float32)
                + lax.dot_general(vlo, es_hi[tk], dn, preferred_element_type=jnp.float32)
                + lax.dot_general(vhi, es_lo[tk], dn, preferred_element_type=jnp.float32))

    def dot_pE(p, tk):
        phi, plo = split(p)
        return (jnp.dot(phi, es_hi[tk], preferred_element_type=jnp.float32)
                + jnp.dot(plo, es_hi[tk], preferred_element_type=jnp.float32)
                + jnp.dot(phi, es_lo[tk], preferred_element_type=jnp.float32))

    q = q_ref[0]
    u = q
    logits = None
    for tk in range(HOPS):
        logits = dot_vE(u, tk)
        p = jax.nn.softmax(logits, axis=-1)
        u = u + dot_pE(p, tk + 1)
    gp = jax.nn.sigmoid(logits)

    u2 = q
    p2 = None
    l2 = None
    for tk in range(HOPS):
        l2 = dot_vE(u2, tk) * gp
        p2 = jax.nn.softmax(l2, axis=-1)
        u2 = u2 + dot_pE(p2 * gp, tk + 1)
    p2_ref[0] = p2
    l2_ref[0] = l2


def _tc_hops(kb, dl, e, dh, q, interpret=False):
    return pl.pallas_call(
        _tc_body,
        grid=(B,),
        in_specs=[
            pl.BlockSpec(memory_space=pltpu.SMEM),
            pl.BlockSpec(memory_space=pltpu.SMEM),
            pl.BlockSpec((1, NT, M, D), lambda b: (b, 0, 0, 0)),
            pl.BlockSpec((1, S, D), lambda b: (b, 0, 0)),
            pl.BlockSpec((1, 1, D), lambda b: (b, 0, 0)),
        ],
        out_specs=[pl.BlockSpec((1, 1, M), lambda b: (b, 0, 0)),
                   pl.BlockSpec((1, 1, M), lambda b: (b, 0, 0))],
        out_shape=[jax.ShapeDtypeStruct((B, 1, M), jnp.float32),
                   jax.ShapeDtypeStruct((B, 1, M), jnp.float32)],
        interpret=interpret,
    )(kb, dl, e, dh, q.reshape(B, 1, D))


def kernel(memory, kb_len, dialog_len, dialog_hidden, query, C):
    mem_flat = memory.reshape(-1).astype(jnp.int32)
    e = _sc_embed(mem_flat, C)
    probs2, logits2 = _tc_hops(kb_len.astype(jnp.int32), dialog_len.astype(jnp.int32),
                               e, dialog_hidden, query)
    return (probs2.reshape(B, M), logits2.reshape(B, M))


# speedup vs baseline: 1.3302x; 1.0177x over previous
import functools

import jax
import jax.numpy as jnp
from jax import lax
from jax.experimental import pallas as pl
from jax.experimental.pallas import tpu as pltpu
from jax.experimental.pallas import tpu_sc as plsc

VOCAB = 100000
D = 128
HOPS = 3
B, M, L, S = 128, 1024, 4, 200

NW = 32
CHUNK_O = 32
CHUNK_I = CHUNK_O * L
ROWS_PER_W = (B * M) // NW
N_CHUNKS = ROWS_PER_W // CHUNK_O


NT = HOPS + 1


def _sc_embed(mem_flat, c_tables):
    mesh = plsc.VectorSubcoreMesh(core_axis_name="c", subcore_axis_name="s")

    @functools.partial(
        pl.kernel,
        out_type=jax.ShapeDtypeStruct((B, NT, M, D), jnp.float32),
        mesh=mesh,
        scratch_types=[
            pltpu.VMEM((2, CHUNK_I), jnp.int32),
            pltpu.VMEM((2, CHUNK_I, D), jnp.float32),
            pltpu.VMEM((2, NT, CHUNK_O, D), jnp.float32),
            pltpu.SemaphoreType.DMA,
            pltpu.SemaphoreType.DMA,
            pltpu.SemaphoreType.DMA,
            pltpu.SemaphoreType.DMA,
            pltpu.SemaphoreType.DMA,
            pltpu.SemaphoreType.DMA,
        ],
    )
    def k(mem_hbm, c_hbm, out_hbm, idx_v, rows_v, out_v,
          isem0, isem1, gsem0, gsem1, osem0, osem1):
        wid = lax.axis_index("s") * 2 + lax.axis_index("c")
        row0 = wid * ROWS_PER_W
        isems = (isem0, isem1)
        gsems = (gsem0, gsem1)
        osems = (osem0, osem1)

        def idx_load(ci, buf, sem):
            return pltpu.async_copy(
                mem_hbm.at[pl.ds((row0 + ci * CHUNK_O) * L, CHUNK_I)],
                idx_v.at[buf], sem)

        def gather(cbuf, tk, gbuf):
            return pltpu.async_copy(c_hbm.at[tk].at[idx_v.at[cbuf]],
                                    rows_v.at[gbuf], gsems[gbuf])

        def wait_gather(gbuf):
            pltpu.make_async_copy(c_hbm.at[0].at[idx_v.at[0]],
                                  rows_v.at[gbuf], gsems[gbuf]).wait()

        idx_load(0, 0, isems[0]).wait()
        gather(0, 0, 0)

        def pair_body(cj, carry):
            for par in range(2):
                ci = 2 * cj + par
                out_row = row0 + ci * CHUNK_O
                b = out_row // M
                m = lax.rem(out_row, M)
                obuf = par
                cbuf = par

                @pl.when(ci + 1 < N_CHUNKS)
                def _(ci=ci, cbuf=cbuf):
                    idx_load(ci + 1, 1 - cbuf, isems[1 - cbuf])

                @pl.when(ci >= 2)
                def _(obuf=obuf):
                    for tk in range(NT):
                        pltpu.make_async_copy(
                            out_v.at[obuf, tk],
                            out_hbm.at[0, tk, pl.ds(0, CHUNK_O)],
                            osems[obuf]).wait()

                for tk in range(NT):
                    gbuf = tk % 2
                    if tk < NT - 1:
                        gather(cbuf, tk + 1, 1 - gbuf)
                    else:
                        @pl.when(ci + 1 < N_CHUNKS)
                        def _(cbuf=cbuf, gbuf=gbuf):
                            pltpu.make_async_copy(
                                mem_hbm.at[pl.ds(0, CHUNK_I)],
                                idx_v.at[1 - cbuf], isems[1 - cbuf]).wait()
                            gather(1 - cbuf, 0, 1 - gbuf)
                    wait_gather(gbuf)

                    def sumrow(i, acc, gbuf=gbuf, tk=tk, obuf=obuf):
                        for g in range(D // 16):
                            sl = pl.ds(g * 16, 16)
                            out_v[obuf, tk, i, sl] = (
                                rows_v[gbuf, L * i, sl] + rows_v[gbuf, L * i + 1, sl]
                                + rows_v[gbuf, L * i + 2, sl] + rows_v[gbuf, L * i + 3, sl])
                        return acc
                    lax.fori_loop(0, CHUNK_O, sumrow, 0, unroll=2)

                    pltpu.async_copy(out_v.at[obuf, tk],
                                     out_hbm.at[b, tk, pl.ds(m, CHUNK_O)],
                                     osems[obuf])
            return carry

        lax.fori_loop(0, N_CHUNKS // 2, pair_body, 0)

        for obuf in range(2):
            for tk in range(NT):
                pltpu.make_async_copy(
                    out_v.at[obuf, tk],
                    out_hbm.at[0, tk, pl.ds(0, CHUNK_O)],
                    osems[obuf]).wait()

    return k(mem_flat, c_tables)


BB = 4


def _tc_body(kb_ref, dl_ref, e_ref, dh_ref, q_ref, p2_ref, l2_ref):
    for bb in range(BB):
        _tc_one(pl.program_id(0) * BB + bb, bb,
                kb_ref, dl_ref, e_ref, dh_ref, q_ref, p2_ref, l2_ref)


def _tc_one(bi, bb, kb_ref, dl_ref, e_ref, dh_ref, q_ref, p2_ref, l2_ref):
    kbv = kb_ref[bi]
    dlv = dl_ref[bi]
    js = lax.broadcasted_iota(jnp.int32, (S, 1), 0)
    hid = dh_ref[bb] * (js < dlv).astype(jnp.float32)
    for tk in range(NT):
        e_ref[bb, tk, pl.ds(kbv, S)] = e_ref[bb, tk, pl.ds(kbv, S)] + hid

    es_hi = []
    es_lo = []
    for tk in range(NT):
        ef = e_ref[bb, tk]
        ehi = ef.astype(jnp.bfloat16)
        es_hi.append(ehi)
        es_lo.append((ef - ehi.astype(jnp.float32)).astype(jnp.bfloat16))

    def split(v):
        vhi = v.astype(jnp.bfloat16)
        return vhi, (v - vhi.astype(jnp.float32)).astype(jnp.bfloat16)

    dn = (((1,), (1,)), ((), ()))

    def dot_vE(v, tk):
        vhi, vlo = split(v)
        return (lax.dot_general(vhi, es_hi[tk], dn, preferred_element_type=jnp.float32)
                + lax.dot_general(vlo, es_hi[tk], dn, preferred_element_type=jnp.float32)
                + lax.dot_general(vhi, es_lo[tk], dn, preferred_element_type=jnp.float32))

    def dot_pE(p, tk):
        phi, plo = split(p)
        return (jnp.dot(phi, es_hi[tk], preferred_element_type=jnp.float32)
                + jnp.dot(plo, es_hi[tk], preferred_element_type=jnp.float32)
                + jnp.dot(phi, es_lo[tk], preferred_element_type=jnp.float32))

    q = q_ref[bb]
    u = q
    logits = None
    for tk in range(HOPS):
        logits = dot_vE(u, tk)
        p = jax.nn.softmax(logits, axis=-1)
        u = u + dot_pE(p, tk + 1)
    gp = jax.nn.sigmoid(logits)

    u2 = q
    p2 = None
    l2 = None
    for tk in range(HOPS):
        l2 = dot_vE(u2, tk) * gp
        p2 = jax.nn.softmax(l2, axis=-1)
        u2 = u2 + dot_pE(p2 * gp, tk + 1)
    p2_ref[bb] = p2
    l2_ref[bb] = l2


def _tc_hops(kb, dl, e, dh, q, interpret=False):
    return pl.pallas_call(
        _tc_body,
        grid=(B // BB,),
        in_specs=[
            pl.BlockSpec(memory_space=pltpu.SMEM),
            pl.BlockSpec(memory_space=pltpu.SMEM),
            pl.BlockSpec((BB, NT, M, D), lambda b: (b, 0, 0, 0)),
            pl.BlockSpec((BB, S, D), lambda b: (b, 0, 0)),
            pl.BlockSpec((BB, 1, D), lambda b: (b, 0, 0)),
        ],
        out_specs=[pl.BlockSpec((BB, 1, M), lambda b: (b, 0, 0)),
                   pl.BlockSpec((BB, 1, M), lambda b: (b, 0, 0))],
        out_shape=[jax.ShapeDtypeStruct((B, 1, M), jnp.float32),
                   jax.ShapeDtypeStruct((B, 1, M), jnp.float32)],
        interpret=interpret,
    )(kb, dl, e, dh, q.reshape(B, 1, D))


def kernel(memory, kb_len, dialog_len, dialog_hidden, query, C):
    mem_flat = memory.reshape(-1).astype(jnp.int32)
    e = _sc_embed(mem_flat, C)
    probs2, logits2 = _tc_hops(kb_len.astype(jnp.int32), dialog_len.astype(jnp.int32),
                               e, dialog_hidden, query)
    return (probs2.reshape(B, M), logits2.reshape(B, M))


# speedup vs baseline: 1.5575x; 1.1709x over previous
import functools

import jax
import jax.numpy as jnp
from jax import lax
from jax.experimental import pallas as pl
from jax.experimental.pallas import tpu as pltpu
from jax.experimental.pallas import tpu_sc as plsc

VOCAB = 100000
D = 128
HOPS = 3
B, M, L, S = 128, 1024, 4, 200

NW = 32
CHUNK_O = 32
CHUNK_I = CHUNK_O * L
NSPLIT = 2
PB = B // NSPLIT
ROWS_PER_W = (PB * M) // NW
N_CHUNKS = ROWS_PER_W // CHUNK_O


NT = HOPS + 1


def _sc_embed(mem_flat, c_tables):
    mesh = plsc.VectorSubcoreMesh(core_axis_name="c", subcore_axis_name="s")

    @functools.partial(
        pl.kernel,
        out_type=jax.ShapeDtypeStruct((PB, NT, M, D), jnp.float32),
        mesh=mesh,
        scratch_types=[
            pltpu.VMEM((2, CHUNK_I), jnp.int32),
            pltpu.VMEM((2, CHUNK_I, D), jnp.float32),
            pltpu.VMEM((2, NT, CHUNK_O, D), jnp.float32),
            pltpu.SemaphoreType.DMA,
            pltpu.SemaphoreType.DMA,
            pltpu.SemaphoreType.DMA,
            pltpu.SemaphoreType.DMA,
            pltpu.SemaphoreType.DMA,
            pltpu.SemaphoreType.DMA,
        ],
    )
    def k(mem_hbm, c_hbm, out_hbm, idx_v, rows_v, out_v,
          isem0, isem1, gsem0, gsem1, osem0, osem1):
        wid = lax.axis_index("s") * 2 + lax.axis_index("c")
        row0 = wid * ROWS_PER_W
        isems = (isem0, isem1)
        gsems = (gsem0, gsem1)
        osems = (osem0, osem1)

        def idx_load(ci, buf, sem):
            return pltpu.async_copy(
                mem_hbm.at[pl.ds((row0 + ci * CHUNK_O) * L, CHUNK_I)],
                idx_v.at[buf], sem)

        def gather(cbuf, tk, gbuf):
            return pltpu.async_copy(c_hbm.at[tk].at[idx_v.at[cbuf]],
                                    rows_v.at[gbuf], gsems[gbuf])

        def wait_gather(gbuf):
            pltpu.make_async_copy(c_hbm.at[0].at[idx_v.at[0]],
                                  rows_v.at[gbuf], gsems[gbuf]).wait()

        idx_load(0, 0, isems[0]).wait()
        gather(0, 0, 0)

        def pair_body(cj, carry):
            for par in range(2):
                ci = 2 * cj + par
                out_row = row0 + ci * CHUNK_O
                b = out_row // M
                m = lax.rem(out_row, M)
                obuf = par
                cbuf = par

                @pl.when(ci + 1 < N_CHUNKS)
                def _(ci=ci, cbuf=cbuf):
                    idx_load(ci + 1, 1 - cbuf, isems[1 - cbuf])

                @pl.when(ci >= 2)
                def _(obuf=obuf):
                    for tk in range(NT):
                        pltpu.make_async_copy(
                            out_v.at[obuf, tk],
                            out_hbm.at[0, tk, pl.ds(0, CHUNK_O)],
                            osems[obuf]).wait()

                for tk in range(NT):
                    gbuf = tk % 2
                    if tk < NT - 1:
                        gather(cbuf, tk + 1, 1 - gbuf)
                    else:
                        @pl.when(ci + 1 < N_CHUNKS)
                        def _(cbuf=cbuf, gbuf=gbuf):
                            pltpu.make_async_copy(
                                mem_hbm.at[pl.ds(0, CHUNK_I)],
                                idx_v.at[1 - cbuf], isems[1 - cbuf]).wait()
                            gather(1 - cbuf, 0, 1 - gbuf)
                    wait_gather(gbuf)

                    def sumrow(i, acc, gbuf=gbuf, tk=tk, obuf=obuf):
                        for g in range(D // 16):
                            sl = pl.ds(g * 16, 16)
                            out_v[obuf, tk, i, sl] = (
                                rows_v[gbuf, L * i, sl] + rows_v[gbuf, L * i + 1, sl]
                                + rows_v[gbuf, L * i + 2, sl] + rows_v[gbuf, L * i + 3, sl])
                        return acc
                    lax.fori_loop(0, CHUNK_O, sumrow, 0, unroll=2)

                    pltpu.async_copy(out_v.at[obuf, tk],
                                     out_hbm.at[b, tk, pl.ds(m, CHUNK_O)],
                                     osems[obuf])
            return carry

        lax.fori_loop(0, N_CHUNKS // 2, pair_body, 0)

        for obuf in range(2):
            for tk in range(NT):
                pltpu.make_async_copy(
                    out_v.at[obuf, tk],
                    out_hbm.at[0, tk, pl.ds(0, CHUNK_O)],
                    osems[obuf]).wait()

    return k(mem_flat, c_tables)


BB = 4


def _tc_body(kb_ref, dl_ref, e_ref, dh_ref, q_ref, p2_ref, l2_ref):
    for bb in range(BB):
        _tc_one(pl.program_id(0) * BB + bb, bb,
                kb_ref, dl_ref, e_ref, dh_ref, q_ref, p2_ref, l2_ref)


def _tc_one(bi, bb, kb_ref, dl_ref, e_ref, dh_ref, q_ref, p2_ref, l2_ref):
    kbv = kb_ref[bi]
    dlv = dl_ref[bi]
    js = lax.broadcasted_iota(jnp.int32, (S, 1), 0)
    hid = dh_ref[bb] * (js < dlv).astype(jnp.float32)
    for tk in range(NT):
        e_ref[bb, tk, pl.ds(kbv, S)] = e_ref[bb, tk, pl.ds(kbv, S)] + hid

    es_hi = []
    es_lo = []
    for tk in range(NT):
        ef = e_ref[bb, tk]
        ehi = ef.astype(jnp.bfloat16)
        es_hi.append(ehi)
        es_lo.append((ef - ehi.astype(jnp.float32)).astype(jnp.bfloat16))

    def split(v):
        vhi = v.astype(jnp.bfloat16)
        return vhi, (v - vhi.astype(jnp.float32)).astype(jnp.bfloat16)

    dn = (((1,), (1,)), ((), ()))

    def dot_vE(v, tk):
        vhi, vlo = split(v)
        return (lax.dot_general(vhi, es_hi[tk], dn, preferred_element_type=jnp.float32)
                + lax.dot_general(vlo, es_hi[tk], dn, preferred_element_type=jnp.float32)
                + lax.dot_general(vhi, es_lo[tk], dn, preferred_element_type=jnp.float32))

    def dot_pE(p, tk):
        phi, plo = split(p)
        return (jnp.dot(phi, es_hi[tk], preferred_element_type=jnp.float32)
                + jnp.dot(plo, es_hi[tk], preferred_element_type=jnp.float32)
                + jnp.dot(phi, es_lo[tk], preferred_element_type=jnp.float32))

    q = q_ref[bb]
    u = q
    logits = None
    for tk in range(HOPS):
        logits = dot_vE(u, tk)
        p = jax.nn.softmax(logits, axis=-1)
        u = u + dot_pE(p, tk + 1)
    gp = jax.nn.sigmoid(logits)

    u2 = q
    p2 = None
    l2 = None
    for tk in range(HOPS):
        l2 = dot_vE(u2, tk) * gp
        p2 = jax.nn.softmax(l2, axis=-1)
        u2 = u2 + dot_pE(p2 * gp, tk + 1)
    p2_ref[bb] = p2
    l2_ref[bb] = l2


def _tc_hops(kb, dl, e, dh, q, interpret=False):
    nb = e.shape[0]
    return pl.pallas_call(
        _tc_body,
        grid=(nb // BB,),
        in_specs=[
            pl.BlockSpec(memory_space=pltpu.SMEM),
            pl.BlockSpec(memory_space=pltpu.SMEM),
            pl.BlockSpec((BB, NT, M, D), lambda b: (b, 0, 0, 0)),
            pl.BlockSpec((BB, S, D), lambda b: (b, 0, 0)),
            pl.BlockSpec((BB, 1, D), lambda b: (b, 0, 0)),
        ],
        out_specs=[pl.BlockSpec((BB, 1, M), lambda b: (b, 0, 0)),
                   pl.BlockSpec((BB, 1, M), lambda b: (b, 0, 0))],
        out_shape=[jax.ShapeDtypeStruct((nb, 1, M), jnp.float32),
                   jax.ShapeDtypeStruct((nb, 1, M), jnp.float32)],
        interpret=interpret,
    )(kb, dl, e, dh, q.reshape(nb, 1, D))


def kernel(memory, kb_len, dialog_len, dialog_hidden, query, C):
    kb = kb_len.astype(jnp.int32)
    dl = dialog_len.astype(jnp.int32)
    es = [_sc_embed(memory[i * PB:(i + 1) * PB].reshape(-1).astype(jnp.int32), C)
          for i in range(NSPLIT)]
    outs = [_tc_hops(kb[i * PB:(i + 1) * PB], dl[i * PB:(i + 1) * PB], es[i],
                     dialog_hidden[i * PB:(i + 1) * PB],
                     query[i * PB:(i + 1) * PB])
            for i in range(NSPLIT)]
    probs2 = jnp.concatenate([o[0].reshape(PB, M) for o in outs], axis=0)
    logits2 = jnp.concatenate([o[1].reshape(PB, M) for o in outs], axis=0)
    return (probs2, logits2)


# speedup vs baseline: 1.5584x; 1.0006x over previous
import functools

import jax
import jax.numpy as jnp
from jax import lax
from jax.experimental import pallas as pl
from jax.experimental.pallas import tpu as pltpu
from jax.experimental.pallas import tpu_sc as plsc

VOCAB = 100000
D = 128
HOPS = 3
B, M, L, S = 128, 1024, 4, 200

NW = 32
CHUNK_O = 32
CHUNK_I = CHUNK_O * L
NSPLIT = 2
PB = B // NSPLIT
ROWS_PER_W = (PB * M) // NW
N_CHUNKS = ROWS_PER_W // CHUNK_O


NT = HOPS + 1


def _sc_embed(mem_flat, c_tables):
    mesh = plsc.VectorSubcoreMesh(core_axis_name="c", subcore_axis_name="s")

    @functools.partial(
        pl.kernel,
        out_type=jax.ShapeDtypeStruct((PB, NT, M, D), jnp.float32),
        mesh=mesh,
        scratch_types=[
            pltpu.VMEM((2, CHUNK_I), jnp.int32),
            pltpu.VMEM((2, CHUNK_I, D), jnp.float32),
            pltpu.VMEM((2, NT, CHUNK_O, D), jnp.float32),
            pltpu.SemaphoreType.DMA,
            pltpu.SemaphoreType.DMA,
            pltpu.SemaphoreType.DMA,
            pltpu.SemaphoreType.DMA,
            pltpu.SemaphoreType.DMA,
            pltpu.SemaphoreType.DMA,
        ],
    )
    def k(mem_hbm, c_hbm, out_hbm, idx_v, rows_v, out_v,
          isem0, isem1, gsem0, gsem1, osem0, osem1):
        wid = lax.axis_index("s") * 2 + lax.axis_index("c")
        row0 = wid * ROWS_PER_W
        isems = (isem0, isem1)
        gsems = (gsem0, gsem1)
        osems = (osem0, osem1)

        def idx_load(ci, buf, sem):
            return pltpu.async_copy(
                mem_hbm.at[pl.ds((row0 + ci * CHUNK_O) * L, CHUNK_I)],
                idx_v.at[buf], sem)

        def gather(cbuf, tk, gbuf):
            return pltpu.async_copy(c_hbm.at[tk].at[idx_v.at[cbuf]],
                                    rows_v.at[gbuf], gsems[gbuf])

        def wait_gather(gbuf):
            pltpu.make_async_copy(c_hbm.at[0].at[idx_v.at[0]],
                                  rows_v.at[gbuf], gsems[gbuf]).wait()

        idx_load(0, 0, isems[0]).wait()
        gather(0, 0, 0)

        def pair_body(cj, carry):
            for par in range(2):
                ci = 2 * cj + par
                out_row = row0 + ci * CHUNK_O
                b = out_row // M
                m = lax.rem(out_row, M)
                obuf = par
                cbuf = par

                @pl.when(ci + 1 < N_CHUNKS)
                def _(ci=ci, cbuf=cbuf):
                    idx_load(ci + 1, 1 - cbuf, isems[1 - cbuf])

                @pl.when(ci >= 2)
                def _(obuf=obuf):
                    for tk in range(NT):
                        pltpu.make_async_copy(
                            out_v.at[obuf, tk],
                            out_hbm.at[0, tk, pl.ds(0, CHUNK_O)],
                            osems[obuf]).wait()

                for tk in range(NT):
                    gbuf = tk % 2
                    if tk < NT - 1:
                        gather(cbuf, tk + 1, 1 - gbuf)
                    else:
                        @pl.when(ci + 1 < N_CHUNKS)
                        def _(cbuf=cbuf, gbuf=gbuf):
                            pltpu.make_async_copy(
                                mem_hbm.at[pl.ds(0, CHUNK_I)],
                                idx_v.at[1 - cbuf], isems[1 - cbuf]).wait()
                            gather(1 - cbuf, 0, 1 - gbuf)
                    wait_gather(gbuf)

                    def sumrow(i, acc, gbuf=gbuf, tk=tk, obuf=obuf):
                        for g in range(D // 16):
                            sl = pl.ds(g * 16, 16)
                            out_v[obuf, tk, i, sl] = (
                                rows_v[gbuf, L * i, sl] + rows_v[gbuf, L * i + 1, sl]
                                + rows_v[gbuf, L * i + 2, sl] + rows_v[gbuf, L * i + 3, sl])
                        return acc
                    lax.fori_loop(0, CHUNK_O, sumrow, 0, unroll=2)

                    pltpu.async_copy(out_v.at[obuf, tk],
                                     out_hbm.at[b, tk, pl.ds(m, CHUNK_O)],
                                     osems[obuf])
            return carry

        lax.fori_loop(0, N_CHUNKS // 2, pair_body, 0)

        for obuf in range(2):
            for tk in range(NT):
                pltpu.make_async_copy(
                    out_v.at[obuf, tk],
                    out_hbm.at[0, tk, pl.ds(0, CHUNK_O)],
                    osems[obuf]).wait()

    return k(mem_flat, c_tables)


BB = 4


def _tc_body(kb_ref, dl_ref, e_ref, dh_ref, q_ref, p2_ref, l2_ref):
    for bb in range(BB):
        _tc_one(pl.program_id(0) * BB + bb, bb,
                kb_ref, dl_ref, e_ref, dh_ref, q_ref, p2_ref, l2_ref)


def _tc_one(bi, bb, kb_ref, dl_ref, e_ref, dh_ref, q_ref, p2_ref, l2_ref):
    kbv = kb_ref[bi]
    dlv = dl_ref[bi]
    js = lax.broadcasted_iota(jnp.int32, (M, 1), 0)
    hid = dh_ref[bb] * ((js[:S] < dlv).astype(jnp.float32))
    hid_full = jnp.concatenate([hid, jnp.zeros((M - S, D), jnp.float32)], axis=0)
    hid_full = pltpu.roll(hid_full, kbv, 0)

    es_hi = []
    es_lo = []
    for tk in range(NT):
        ef = e_ref[bb, tk] + hid_full
        ehi = ef.astype(jnp.bfloat16)
        es_hi.append(ehi)
        es_lo.append((ef - ehi.astype(jnp.float32)).astype(jnp.bfloat16))

    def split(v):
        vhi = v.astype(jnp.bfloat16)
        return vhi, (v - vhi.astype(jnp.float32)).astype(jnp.bfloat16)

    dn = (((1,), (1,)), ((), ()))

    def dot_vE(v, tk):
        vhi, vlo = split(v)
        return (lax.dot_general(vhi, es_hi[tk], dn, preferred_element_type=jnp.float32)
                + lax.dot_general(vlo, es_hi[tk], dn, preferred_element_type=jnp.float32)
                + lax.dot_general(vhi, es_lo[tk], dn, preferred_element_type=jnp.float32))

    def dot_pE(p, tk):
        phi, plo = split(p)
        return (jnp.dot(phi, es_hi[tk], preferred_element_type=jnp.float32)
                + jnp.dot(plo, es_hi[tk], preferred_element_type=jnp.float32)
                + jnp.dot(phi, es_lo[tk], preferred_element_type=jnp.float32))

    q = q_ref[bb]
    u = q
    logits = None
    for tk in range(HOPS):
        logits = dot_vE(u, tk)
        p = jax.nn.softmax(logits, axis=-1)
        u = u + dot_pE(p, tk + 1)
    gp = jax.nn.sigmoid(logits)

    u2 = q
    p2 = None
    l2 = None
    for tk in range(HOPS):
        l2 = dot_vE(u2, tk) * gp
        p2 = jax.nn.softmax(l2, axis=-1)
        u2 = u2 + dot_pE(p2 * gp, tk + 1)
    p2_ref[bb] = p2
    l2_ref[bb] = l2


def _tc_hops(kb, dl, e, dh, q, interpret=False):
    nb = e.shape[0]
    return pl.pallas_call(
        _tc_body,
        grid=(nb // BB,),
        in_specs=[
            pl.BlockSpec(memory_space=pltpu.SMEM),
            pl.BlockSpec(memory_space=pltpu.SMEM),
            pl.BlockSpec((BB, NT, M, D), lambda b: (b, 0, 0, 0)),
            pl.BlockSpec((BB, S, D), lambda b: (b, 0, 0)),
            pl.BlockSpec((BB, 1, D), lambda b: (b, 0, 0)),
        ],
        out_specs=[pl.BlockSpec((BB, 1, M), lambda b: (b, 0, 0)),
                   pl.BlockSpec((BB, 1, M), lambda b: (b, 0, 0))],
        out_shape=[jax.ShapeDtypeStruct((nb, 1, M), jnp.float32),
                   jax.ShapeDtypeStruct((nb, 1, M), jnp.float32)],
        interpret=interpret,
    )(kb, dl, e, dh, q.reshape(nb, 1, D))


def kernel(memory, kb_len, dialog_len, dialog_hidden, query, C):
    kb = kb_len.astype(jnp.int32)
    dl = dialog_len.astype(jnp.int32)
    es = [_sc_embed(memory[i * PB:(i + 1) * PB].reshape(-1).astype(jnp.int32), C)
          for i in range(NSPLIT)]
    outs = [_tc_hops(kb[i * PB:(i + 1) * PB], dl[i * PB:(i + 1) * PB], es[i],
                     dialog_hidden[i * PB:(i + 1) * PB],
                     query[i * PB:(i + 1) * PB])
            for i in range(NSPLIT)]
    probs2 = jnp.concatenate([o[0].reshape(PB, M) for o in outs], axis=0)
    logits2 = jnp.concatenate([o[1].reshape(PB, M) for o in outs], axis=0)
    return (probs2, logits2)


# speedup vs baseline: 1.6743x; 1.0743x over previous
import functools

import jax
import jax.numpy as jnp
from jax import lax
from jax.experimental import pallas as pl
from jax.experimental.pallas import tpu as pltpu
from jax.experimental.pallas import tpu_sc as plsc

VOCAB = 100000
D = 128
HOPS = 3
B, M, L, S = 128, 1024, 4, 200

NW = 32
CHUNK_O = 32
CHUNK_I = CHUNK_O * L
NSPLIT = 4
PB = B // NSPLIT
ROWS_PER_W = (PB * M) // NW
N_CHUNKS = ROWS_PER_W // CHUNK_O


NT = HOPS + 1


def _sc_embed(mem_flat, c_tables):
    mesh = plsc.VectorSubcoreMesh(core_axis_name="c", subcore_axis_name="s")

    @functools.partial(
        pl.kernel,
        out_type=jax.ShapeDtypeStruct((PB, NT, M, D), jnp.float32),
        mesh=mesh,
        scratch_types=[
            pltpu.VMEM((2, CHUNK_I), jnp.int32),
            pltpu.VMEM((2, CHUNK_I, D), jnp.float32),
            pltpu.VMEM((2, NT, CHUNK_O, D), jnp.float32),
            pltpu.SemaphoreType.DMA,
            pltpu.SemaphoreType.DMA,
            pltpu.SemaphoreType.DMA,
            pltpu.SemaphoreType.DMA,
            pltpu.SemaphoreType.DMA,
            pltpu.SemaphoreType.DMA,
        ],
    )
    def k(mem_hbm, c_hbm, out_hbm, idx_v, rows_v, out_v,
          isem0, isem1, gsem0, gsem1, osem0, osem1):
        wid = lax.axis_index("s") * 2 + lax.axis_index("c")
        row0 = wid * ROWS_PER_W
        isems = (isem0, isem1)
        gsems = (gsem0, gsem1)
        osems = (osem0, osem1)

        def idx_load(ci, buf, sem):
            return pltpu.async_copy(
                mem_hbm.at[pl.ds((row0 + ci * CHUNK_O) * L, CHUNK_I)],
                idx_v.at[buf], sem)

        def gather(cbuf, tk, gbuf):
            return pltpu.async_copy(c_hbm.at[tk].at[idx_v.at[cbuf]],
                                    rows_v.at[gbuf], gsems[gbuf])

        def wait_gather(gbuf):
            pltpu.make_async_copy(c_hbm.at[0].at[idx_v.at[0]],
                                  rows_v.at[gbuf], gsems[gbuf]).wait()

        idx_load(0, 0, isems[0]).wait()
        gather(0, 0, 0)

        def pair_body(cj, carry):
            for par in range(2):
                ci = 2 * cj + par
                out_row = row0 + ci * CHUNK_O
                b = out_row // M
                m = lax.rem(out_row, M)
                obuf = par
                cbuf = par

                @pl.when(ci + 1 < N_CHUNKS)
                def _(ci=ci, cbuf=cbuf):
                    idx_load(ci + 1, 1 - cbuf, isems[1 - cbuf])

                @pl.when(ci >= 2)
                def _(obuf=obuf):
                    for tk in range(NT):
                        pltpu.make_async_copy(
                            out_v.at[obuf, tk],
                            out_hbm.at[0, tk, pl.ds(0, CHUNK_O)],
                            osems[obuf]).wait()

                for tk in range(NT):
                    gbuf = tk % 2
                    if tk < NT - 1:
                        gather(cbuf, tk + 1, 1 - gbuf)
                    else:
                        @pl.when(ci + 1 < N_CHUNKS)
                        def _(cbuf=cbuf, gbuf=gbuf):
                            pltpu.make_async_copy(
                                mem_hbm.at[pl.ds(0, CHUNK_I)],
                                idx_v.at[1 - cbuf], isems[1 - cbuf]).wait()
                            gather(1 - cbuf, 0, 1 - gbuf)
                    wait_gather(gbuf)

                    def sumrow(i, acc, gbuf=gbuf, tk=tk, obuf=obuf):
                        for g in range(D // 16):
                            sl = pl.ds(g * 16, 16)
                            out_v[obuf, tk, i, sl] = (
                                rows_v[gbuf, L * i, sl] + rows_v[gbuf, L * i + 1, sl]
                                + rows_v[gbuf, L * i + 2, sl] + rows_v[gbuf, L * i + 3, sl])
                        return acc
                    lax.fori_loop(0, CHUNK_O, sumrow, 0, unroll=2)

                    pltpu.async_copy(out_v.at[obuf, tk],
                                     out_hbm.at[b, tk, pl.ds(m, CHUNK_O)],
                                     osems[obuf])
            return carry

        lax.fori_loop(0, N_CHUNKS // 2, pair_body, 0)

        for obuf in range(2):
            for tk in range(NT):
                pltpu.make_async_copy(
                    out_v.at[obuf, tk],
                    out_hbm.at[0, tk, pl.ds(0, CHUNK_O)],
                    osems[obuf]).wait()

    return k(mem_flat, c_tables)


BB = 4


def _tc_body(kb_ref, dl_ref, e_ref, dh_ref, q_ref, p2_ref, l2_ref):
    for bb in range(BB):
        _tc_one(pl.program_id(0) * BB + bb, bb,
                kb_ref, dl_ref, e_ref, dh_ref, q_ref, p2_ref, l2_ref)


def _tc_one(bi, bb, kb_ref, dl_ref, e_ref, dh_ref, q_ref, p2_ref, l2_ref):
    kbv = kb_ref[bi]
    dlv = dl_ref[bi]
    js = lax.broadcasted_iota(jnp.int32, (M, 1), 0)
    hid = dh_ref[bb] * ((js[:S] < dlv).astype(jnp.float32))
    hid_full = jnp.concatenate([hid, jnp.zeros((M - S, D), jnp.float32)], axis=0)
    hid_full = pltpu.roll(hid_full, kbv, 0)

    es_hi = []
    es_lo = []
    for tk in range(NT):
        ef = e_ref[bb, tk] + hid_full
        ehi = ef.astype(jnp.bfloat16)
        es_hi.append(ehi)
        es_lo.append((ef - ehi.astype(jnp.float32)).astype(jnp.bfloat16))

    def split(v):
        vhi = v.astype(jnp.bfloat16)
        return vhi, (v - vhi.astype(jnp.float32)).astype(jnp.bfloat16)

    dn = (((1,), (1,)), ((), ()))

    def dot_vE(v, tk):
        vhi, vlo = split(v)
        return (lax.dot_general(vhi, es_hi[tk], dn, preferred_element_type=jnp.float32)
                + lax.dot_general(vlo, es_hi[tk], dn, preferred_element_type=jnp.float32)
                + lax.dot_general(vhi, es_lo[tk], dn, preferred_element_type=jnp.float32))

    def dot_pE(p, tk):
        phi, plo = split(p)
        return (jnp.dot(phi, es_hi[tk], preferred_element_type=jnp.float32)
                + jnp.dot(plo, es_hi[tk], preferred_element_type=jnp.float32)
                + jnp.dot(phi, es_lo[tk], preferred_element_type=jnp.float32))

    q = q_ref[bb]
    u = q
    logits = None
    for tk in range(HOPS):
        logits = dot_vE(u, tk)
        p = jax.nn.softmax(logits, axis=-1)
        u = u + dot_pE(p, tk + 1)
    gp = jax.nn.sigmoid(logits)

    u2 = q
    p2 = None
    l2 = None
    for tk in range(HOPS):
        l2 = dot_vE(u2, tk) * gp
        p2 = jax.nn.softmax(l2, axis=-1)
        u2 = u2 + dot_pE(p2 * gp, tk + 1)
    p2_ref[bb] = p2
    l2_ref[bb] = l2


def _tc_hops(kb, dl, e, dh, q, interpret=False):
    nb = e.shape[0]
    return pl.pallas_call(
        _tc_body,
        grid=(nb // BB,),
        in_specs=[
            pl.BlockSpec(memory_space=pltpu.SMEM),
            pl.BlockSpec(memory_space=pltpu.SMEM),
            pl.BlockSpec((BB, NT, M, D), lambda b: (b, 0, 0, 0)),
            pl.BlockSpec((BB, S, D), lambda b: (b, 0, 0)),
            pl.BlockSpec((BB, 1, D), lambda b: (b, 0, 0)),
        ],
        out_specs=[pl.BlockSpec((BB, 1, M), lambda b: (b, 0, 0)),
                   pl.BlockSpec((BB, 1, M), lambda b: (b, 0, 0))],
        out_shape=[jax.ShapeDtypeStruct((nb, 1, M), jnp.float32),
                   jax.ShapeDtypeStruct((nb, 1, M), jnp.float32)],
        interpret=interpret,
    )(kb, dl, e, dh, q.reshape(nb, 1, D))


def kernel(memory, kb_len, dialog_len, dialog_hidden, query, C):
    kb = kb_len.astype(jnp.int32)
    dl = dialog_len.astype(jnp.int32)
    es = [_sc_embed(memory[i * PB:(i + 1) * PB].reshape(-1).astype(jnp.int32), C)
          for i in range(NSPLIT)]
    outs = [_tc_hops(kb[i * PB:(i + 1) * PB], dl[i * PB:(i + 1) * PB], es[i],
                     dialog_hidden[i * PB:(i + 1) * PB],
                     query[i * PB:(i + 1) * PB])
            for i in range(NSPLIT)]
    probs2 = jnp.concatenate([o[0].reshape(PB, M) for o in outs], axis=0)
    logits2 = jnp.concatenate([o[1].reshape(PB, M) for o in outs], axis=0)
    return (probs2, logits2)


# speedup vs baseline: 1.6832x; 1.0053x over previous
import functools

import jax
import jax.numpy as jnp
from jax import lax
from jax.experimental import pallas as pl
from jax.experimental.pallas import tpu as pltpu
from jax.experimental.pallas import tpu_sc as plsc

VOCAB = 100000
D = 128
HOPS = 3
B, M, L, S = 128, 1024, 4, 200

NW = 32
CHUNK_O = 32
CHUNK_I = CHUNK_O * L
NSPLIT = 8
PB = B // NSPLIT
ROWS_PER_W = (PB * M) // NW
N_CHUNKS = ROWS_PER_W // CHUNK_O


NT = HOPS + 1


def _sc_embed(mem_flat, c_tables):
    mesh = plsc.VectorSubcoreMesh(core_axis_name="c", subcore_axis_name="s")

    @functools.partial(
        pl.kernel,
        out_type=jax.ShapeDtypeStruct((PB, NT, M, D), jnp.float32),
        mesh=mesh,
        scratch_types=[
            pltpu.VMEM((2, CHUNK_I), jnp.int32),
            pltpu.VMEM((2, CHUNK_I, D), jnp.float32),
            pltpu.VMEM((2, NT, CHUNK_O, D), jnp.float32),
            pltpu.SemaphoreType.DMA,
            pltpu.SemaphoreType.DMA,
            pltpu.SemaphoreType.DMA,
            pltpu.SemaphoreType.DMA,
            pltpu.SemaphoreType.DMA,
            pltpu.SemaphoreType.DMA,
        ],
    )
    def k(mem_hbm, c_hbm, out_hbm, idx_v, rows_v, out_v,
          isem0, isem1, gsem0, gsem1, osem0, osem1):
        wid = lax.axis_index("s") * 2 + lax.axis_index("c")
        row0 = wid * ROWS_PER_W
        isems = (isem0, isem1)
        gsems = (gsem0, gsem1)
        osems = (osem0, osem1)

        def idx_load(ci, buf, sem):
            return pltpu.async_copy(
                mem_hbm.at[pl.ds((row0 + ci * CHUNK_O) * L, CHUNK_I)],
                idx_v.at[buf], sem)

        def gather(cbuf, tk, gbuf):
            return pltpu.async_copy(c_hbm.at[tk].at[idx_v.at[cbuf]],
                                    rows_v.at[gbuf], gsems[gbuf])

        def wait_gather(gbuf):
            pltpu.make_async_copy(c_hbm.at[0].at[idx_v.at[0]],
                                  rows_v.at[gbuf], gsems[gbuf]).wait()

        idx_load(0, 0, isems[0]).wait()
        gather(0, 0, 0)

        def pair_body(cj, carry):
            for par in range(2):
                ci = 2 * cj + par
                out_row = row0 + ci * CHUNK_O
                b = out_row // M
                m = lax.rem(out_row, M)
                obuf = par
                cbuf = par

                @pl.when(ci + 1 < N_CHUNKS)
                def _(ci=ci, cbuf=cbuf):
                    idx_load(ci + 1, 1 - cbuf, isems[1 - cbuf])

                @pl.when(ci >= 2)
                def _(obuf=obuf):
                    for tk in range(NT):
                        pltpu.make_async_copy(
                            out_v.at[obuf, tk],
                            out_hbm.at[0, tk, pl.ds(0, CHUNK_O)],
                            osems[obuf]).wait()

                for tk in range(NT):
                    gbuf = tk % 2
                    if tk < NT - 1:
                        gather(cbuf, tk + 1, 1 - gbuf)
                    else:
                        @pl.when(ci + 1 < N_CHUNKS)
                        def _(cbuf=cbuf, gbuf=gbuf):
                            pltpu.make_async_copy(
                                mem_hbm.at[pl.ds(0, CHUNK_I)],
                                idx_v.at[1 - cbuf], isems[1 - cbuf]).wait()
                            gather(1 - cbuf, 0, 1 - gbuf)
                    wait_gather(gbuf)

                    def sumrow(i, acc, gbuf=gbuf, tk=tk, obuf=obuf):
                        for g in range(D // 16):
                            sl = pl.ds(g * 16, 16)
                            out_v[obuf, tk, i, sl] = (
                                rows_v[gbuf, L * i, sl] + rows_v[gbuf, L * i + 1, sl]
                                + rows_v[gbuf, L * i + 2, sl] + rows_v[gbuf, L * i + 3, sl])
                        return acc
                    lax.fori_loop(0, CHUNK_O, sumrow, 0, unroll=2)

                    pltpu.async_copy(out_v.at[obuf, tk],
                                     out_hbm.at[b, tk, pl.ds(m, CHUNK_O)],
                                     osems[obuf])
            return carry

        lax.fori_loop(0, N_CHUNKS // 2, pair_body, 0)

        for obuf in range(2):
            for tk in range(NT):
                pltpu.make_async_copy(
                    out_v.at[obuf, tk],
                    out_hbm.at[0, tk, pl.ds(0, CHUNK_O)],
                    osems[obuf]).wait()

    return k(mem_flat, c_tables)


BB = 4


def _tc_body(kb_ref, dl_ref, e_ref, dh_ref, q_ref, p2_ref, l2_ref):
    for bb in range(BB):
        _tc_one(pl.program_id(0) * BB + bb, bb,
                kb_ref, dl_ref, e_ref, dh_ref, q_ref, p2_ref, l2_ref)


def _tc_one(bi, bb, kb_ref, dl_ref, e_ref, dh_ref, q_ref, p2_ref, l2_ref):
    kbv = kb_ref[bi]
    dlv = dl_ref[bi]
    js = lax.broadcasted_iota(jnp.int32, (M, 1), 0)
    hid = dh_ref[bb] * ((js[:S] < dlv).astype(jnp.float32))
    hid_full = jnp.concatenate([hid, jnp.zeros((M - S, D), jnp.float32)], axis=0)
    hid_full = pltpu.roll(hid_full, kbv, 0)

    es_hi = []
    es_lo = []
    for tk in range(NT):
        ef = e_ref[bb, tk] + hid_full
        ehi = ef.astype(jnp.bfloat16)
        es_hi.append(ehi)
        es_lo.append((ef - ehi.astype(jnp.float32)).astype(jnp.bfloat16))

    def split(v):
        vhi = v.astype(jnp.bfloat16)
        return vhi, (v - vhi.astype(jnp.float32)).astype(jnp.bfloat16)

    dn = (((1,), (1,)), ((), ()))

    def dot_vE(v, tk):
        vhi, vlo = split(v)
        return (lax.dot_general(vhi, es_hi[tk], dn, preferred_element_type=jnp.float32)
                + lax.dot_general(vlo, es_hi[tk], dn, preferred_element_type=jnp.float32)
                + lax.dot_general(vhi, es_lo[tk], dn, preferred_element_type=jnp.float32))

    def dot_pE(p, tk):
        phi, plo = split(p)
        return (jnp.dot(phi, es_hi[tk], preferred_element_type=jnp.float32)
                + jnp.dot(plo, es_hi[tk], preferred_element_type=jnp.float32)
                + jnp.dot(phi, es_lo[tk], preferred_element_type=jnp.float32))

    q = q_ref[bb]
    u = q
    logits = None
    for tk in range(HOPS):
        logits = dot_vE(u, tk)
        p = jax.nn.softmax(logits, axis=-1)
        u = u + dot_pE(p, tk + 1)
    gp = jax.nn.sigmoid(logits)

    u2 = q
    p2 = None
    l2 = None
    for tk in range(HOPS):
        l2 = dot_vE(u2, tk) * gp
        p2 = jax.nn.softmax(l2, axis=-1)
        u2 = u2 + dot_pE(p2 * gp, tk + 1)
    p2_ref[bb] = p2
    l2_ref[bb] = l2


def _tc_hops(kb, dl, e, dh, q, interpret=False):
    nb = e.shape[0]
    return pl.pallas_call(
        _tc_body,
        grid=(nb // BB,),
        in_specs=[
            pl.BlockSpec(memory_space=pltpu.SMEM),
            pl.BlockSpec(memory_space=pltpu.SMEM),
            pl.BlockSpec((BB, NT, M, D), lambda b: (b, 0, 0, 0)),
            pl.BlockSpec((BB, S, D), lambda b: (b, 0, 0)),
            pl.BlockSpec((BB, 1, D), lambda b: (b, 0, 0)),
        ],
        out_specs=[pl.BlockSpec((BB, 1, M), lambda b: (b, 0, 0)),
                   pl.BlockSpec((BB, 1, M), lambda b: (b, 0, 0))],
        out_shape=[jax.ShapeDtypeStruct((nb, 1, M), jnp.float32),
                   jax.ShapeDtypeStruct((nb, 1, M), jnp.float32)],
        interpret=interpret,
    )(kb, dl, e, dh, q.reshape(nb, 1, D))


def kernel(memory, kb_len, dialog_len, dialog_hidden, query, C):
    kb = kb_len.astype(jnp.int32)
    dl = dialog_len.astype(jnp.int32)
    es = [_sc_embed(memory[i * PB:(i + 1) * PB].reshape(-1).astype(jnp.int32), C)
          for i in range(NSPLIT)]
    outs = [_tc_hops(kb[i * PB:(i + 1) * PB], dl[i * PB:(i + 1) * PB], es[i],
                     dialog_hidden[i * PB:(i + 1) * PB],
                     query[i * PB:(i + 1) * PB])
            for i in range(NSPLIT)]
    probs2 = jnp.concatenate([o[0].reshape(PB, M) for o in outs], axis=0)
    logits2 = jnp.concatenate([o[1].reshape(PB, M) for o in outs], axis=0)
    return (probs2, logits2)
